# Initial kernel scaffold; baseline (speedup 1.0000x reference)
import jax
import jax.numpy as jnp
from jax.experimental import pallas as pl


def kernel(X, edge_index, edge_attr, matched_car_infra_nodes, W_node, b_node, W_edge, b_edge, Wh, ah, W_out, a_out):
    raise NotImplementedError("write your pallas kernel here")


# speedup vs baseline: 10.3775x; 10.3775x over previous
import functools
import jax
import jax.numpy as jnp
from jax import lax
from jax.experimental import pallas as pl
from jax.experimental.pallas import tpu as pltpu
from jax.experimental.pallas import tpu_sc as plsc

N = 10000
E = 320000
HEADS = 4
OUT = 64
EMBED = 128
K = 80
NS = 16
NPAD = 10240
RPT = NPAD // NS
F32 = jnp.float32
I32 = jnp.int32


def _node_dense_body(x_ref, wn_ref, bn_ref, wcat_ref, bcat_ref, h_ref, s_ref):
    x = jnp.maximum(jnp.dot(x_ref[...], wn_ref[...], preferred_element_type=F32)
                    + bn_ref[...], 0.0)
    h_ref[...] = jnp.dot(x, wcat_ref[...], preferred_element_type=F32)
    s_ref[...] = jnp.dot(x, bcat_ref[...], preferred_element_type=F32)


def _node_dense(X, W_node, bn2d, Wcat, Bcat):
    blk = 2000
    return pl.pallas_call(
        _node_dense_body,
        grid=(N // blk,),
        in_specs=[
            pl.BlockSpec((blk, 128), lambda i: (i, 0)),
            pl.BlockSpec((128, 128), lambda i: (0, 0)),
            pl.BlockSpec((1, 128), lambda i: (0, 0)),
            pl.BlockSpec((128, 256), lambda i: (0, 0)),
            pl.BlockSpec((128, 8), lambda i: (0, 0)),
        ],
        out_specs=[
            pl.BlockSpec((blk, 256), lambda i: (i, 0)),
            pl.BlockSpec((blk, 8), lambda i: (i, 0)),
        ],
        out_shape=[
            jax.ShapeDtypeStruct((N, 256), F32),
            jax.ShapeDtypeStruct((N, 8), F32),
        ],
    )(X, W_node, bn2d, Wcat, Bcat)


def _gamma_body(eaT_ref, weT_ref, be_ref, a3_ref, g_ref):
    t = jnp.maximum(jnp.dot(weT_ref[...], eaT_ref[...], preferred_element_type=F32)
                    + be_ref[...], 0.0)
    g_ref[...] = jnp.dot(a3_ref[...], t, preferred_element_type=F32)


def _gamma(eaT, W_edgeT, be2d, A3p):
    blk = 6400
    return pl.pallas_call(
        _gamma_body,
        grid=(E // blk,),
        in_specs=[
            pl.BlockSpec((16, blk), lambda i: (0, i)),
            pl.BlockSpec((128, 16), lambda i: (0, 0)),
            pl.BlockSpec((128, 1), lambda i: (0, 0)),
            pl.BlockSpec((8, 128), lambda i: (0, 0)),
        ],
        out_specs=pl.BlockSpec((8, blk), lambda i: (0, i)),
        out_shape=jax.ShapeDtypeStruct((8, E), F32),
    )(eaT, W_edgeT, be2d, A3p)


def _zero_ref(ref, nrows, ncols):
    z = jnp.zeros((16,), F32)

    @pl.loop(0, nrows)
    def _(i):
        for q in range(ncols // 16):
            ref[i, pl.ds(q * 16, 16)] = z


def _edge_pass(src_h, tgt_h, gflat_h, stab_h, table_h, u_h, p_h,
               s_tab, src_v, tgt_v, tgtH_v, g0_v, p0_v,
               rows_v, zrow_v, zp_v, pb_v, u_sh, p_sh,
               *, kidx, epc):
    s = lax.axis_index("s")

    c = lax.axis_index("c")
    if kidx is None:
        pltpu.sync_copy(stab_h, s_tab)
    else:
        hg = 2 * kidx + c
        pltpu.sync_copy(stab_h.at[pl.ds(hg * 2 * N, 2 * N)], s_tab)

    _zero_ref(zrow_v, 128, OUT)
    _zero_ref(zp_v, 128, 16)
    for q in range(RPT // 128):
        pltpu.sync_copy(zrow_v, u_sh.at[pl.ds(s * RPT + q * 128, 128), :])
        pltpu.sync_copy(zp_v, p_sh.at[pl.ds(s * RPT + q * 128, 128), :])
    plsc.subcore_barrier()

    if kidx is None:
        base0 = (c * NS + s) * epc
        goff = 4 * E
    else:
        base0 = s * epc
        goff = hg * E
    nchunks = epc // K

    @pl.loop(0, nchunks)
    def _chunk(i):
        base = base0 + i * K
        pltpu.sync_copy(src_h.at[pl.ds(base, K)], src_v)
        pltpu.sync_copy(tgt_h.at[pl.ds(base, K)], tgt_v)
        pltpu.sync_copy(gflat_h.at[pl.ds(goff + base, K)], g0_v)

        for g in range(K // 16):
            sv = src_v[pl.ds(g * 16, 16)]
            tv = tgt_v[pl.ds(g * 16, 16)]
            if kidx is not None:
                tgtH_v[pl.ds(g * 16, 16)] = tv + hg * N
            s1 = plsc.load_gather(s_tab, [sv * 2])
            s2 = plsc.load_gather(s_tab, [tv * 2 + 1])
            gv = g0_v[pl.ds(g * 16, 16)]
            sc = s1 + s2 + gv
            sc = jnp.where(sc >= 0.0, sc, sc * 0.01)
            p0_v[pl.ds(g * 16, 16)] = jnp.exp(sc)

        idx_ref = tgt_v if kidx is None else tgtH_v
        pltpu.sync_copy(table_h.at[idx_ref], rows_v)

        iot = lax.iota(I32, 16)
        for g in range(K // 16):
            pv0 = p0_v[pl.ds(g * 16, 16)]
            for j in range(16):
                jj = g * 16 + j
                b0 = jnp.full((16,), pv0[j], F32)
                pb_v[jj, :] = jnp.where(iot == 0, b0, 0.0)
                for q in range(OUT // 16):
                    rows_v[jj, pl.ds(q * 16, 16)] = (
                        rows_v[jj, pl.ds(q * 16, 16)] * b0)

        pltpu.sync_copy(rows_v, u_sh.at[src_v], add=True)
        pltpu.sync_copy(pb_v, p_sh.at[src_v], add=True)

    plsc.subcore_barrier()

    out_base = c * NPAD + s * RPT
    pltpu.sync_copy(u_sh.at[pl.ds(s * RPT, RPT), :], u_h.at[pl.ds(out_base, RPT), :])
    pltpu.sync_copy(p_sh.at[pl.ds(s * RPT, RPT), :], p_h.at[pl.ds(out_base, RPT), :])


def _make_edge_pass(kidx, epc, tabn):
    mesh = plsc.VectorSubcoreMesh(core_axis_name="c", subcore_axis_name="s")
    body = functools.partial(_edge_pass, kidx=kidx, epc=epc)
    return pl.kernel(
        body,
        out_type=[
            jax.ShapeDtypeStruct((2 * NPAD, OUT), F32),
            jax.ShapeDtypeStruct((2 * NPAD, 16), F32),
        ],
        mesh=mesh,
        scratch_types=[
            pltpu.VMEM((tabn * 2,), F32),
            pltpu.VMEM((K,), I32),
            pltpu.VMEM((K,), I32),
            pltpu.VMEM((K,), I32),
            pltpu.VMEM((K,), F32),
            pltpu.VMEM((K,), F32),
            pltpu.VMEM((K, OUT), F32),
            pltpu.VMEM((128, OUT), F32),
            pltpu.VMEM((128, 16), F32),
            pltpu.VMEM((K, 16), F32),
            pltpu.VMEM_SHARED((NPAD, OUT), F32),
            pltpu.VMEM_SHARED((NPAD, 16), F32),
        ],
        compiler_params=pltpu.CompilerParams(needs_layout_passes=False, use_tc_tiling_on_sc=False),
        name=f"gat_edge_pass_{kidx}",
    )


def _elu(x):
    return jnp.where(x > 0.0, x, jnp.exp(jnp.minimum(x, 0.0)) - 1.0)


def _mid_body(u_ref, p_ref, wout_ref, ab_ref, h2_ref, sb_ref):
    cols = []
    for g in range(4):
        u = u_ref[g]
        d = p_ref[g, :, 0]
        cols.append(_elu(u / (d + 1e-16)[:, None]))
    x2 = _elu(jnp.concatenate(cols, axis=1))
    h2 = jnp.dot(x2, wout_ref[...], preferred_element_type=F32)
    h2_ref[...] = h2
    sb_ref[...] = jnp.dot(h2, ab_ref[...], preferred_element_type=F32)


def _mid(U4, P4, W_out, ab):
    blk = 2048
    return pl.pallas_call(
        _mid_body,
        grid=(NPAD // blk,),
        in_specs=[
            pl.BlockSpec((4, blk, 64), lambda i: (0, i, 0)),
            pl.BlockSpec((4, blk, 16), lambda i: (0, i, 0)),
            pl.BlockSpec((256, 64), lambda i: (0, 0)),
            pl.BlockSpec((64, 2), lambda i: (0, 0)),
        ],
        out_specs=[
            pl.BlockSpec((blk, 64), lambda i: (i, 0)),
            pl.BlockSpec((blk, 2), lambda i: (i, 0)),
        ],
        out_shape=[
            jax.ShapeDtypeStruct((NPAD, 64), F32),
            jax.ShapeDtypeStruct((NPAD, 2), F32),
        ],
    )(U4, P4, W_out, ab)


def _final_body(u_ref, p_ref, o_ref):
    u = u_ref[0] + u_ref[1]
    d = p_ref[0, :, 0] + p_ref[1, :, 0]
    y = _elu(u / (d + 1e-16)[:, None])
    m = jnp.max(y, axis=1, keepdims=True)
    lse = m + jnp.log(jnp.sum(jnp.exp(y - m), axis=1, keepdims=True))
    o_ref[...] = y - lse


def _final(U2, P2):
    blk = 2048
    return pl.pallas_call(
        _final_body,
        grid=(NPAD // blk,),
        in_specs=[
            pl.BlockSpec((2, blk, 64), lambda i: (0, i, 0)),
            pl.BlockSpec((2, blk, 16), lambda i: (0, i, 0)),
        ],
        out_specs=pl.BlockSpec((blk, 64), lambda i: (i, 0)),
        out_shape=jax.ShapeDtypeStruct((NPAD, 64), F32),
    )(U2, P2)


def kernel(X, edge_index, edge_attr, matched_car_infra_nodes, W_node, b_node,
           W_edge, b_edge, Wh, ah, W_out, a_out):
    Wcat = jnp.concatenate([Wh[i] for i in range(HEADS)], axis=1)
    a1 = ah[:, :OUT, 0]
    a2 = ah[:, OUT:2 * OUT, 0]
    a3 = ah[:, 2 * OUT:, 0]
    B1 = jnp.einsum("hio,ho->ih", Wh, a1)
    B2 = jnp.einsum("hio,ho->ih", Wh, a2)
    Bcat = jnp.concatenate([B1, B2], axis=1)
    A3p = jnp.concatenate([a3, a_out[2 * OUT:, :].T,
                           jnp.zeros((3, EMBED), F32)], axis=0)
    ab = jnp.concatenate([a_out[:OUT, :], a_out[OUT:2 * OUT, :]], axis=1)

    src = edge_index[0]
    tgt = edge_index[1]
    eaT = edge_attr.T

    H, S = _node_dense(X, W_node, b_node.reshape(1, 128), Wcat, Bcat)
    G = _gamma(eaT, W_edge.T, b_edge.reshape(128, 1), A3p)
    Gflat = G.reshape(-1)
    Sflat = jnp.stack([jnp.stack([S[:, g], S[:, 4 + g]], axis=1)
                       for g in range(4)], axis=0).reshape(-1)

    H4 = jnp.concatenate([H[:, g * 64:(g + 1) * 64] for g in range(4)], axis=0)

    ep_a = _make_edge_pass(kidx=0, epc=E // NS, tabn=N)
    ep_b = _make_edge_pass(kidx=1, epc=E // NS, tabn=N)
    Ua, Pa = ep_a(src, tgt, Gflat, Sflat, H4)
    Ub, Pb = ep_b(src, tgt, Gflat, Sflat, H4)
    U4 = jnp.concatenate([Ua.reshape(2, NPAD, 64), Ub.reshape(2, NPAD, 64)], 0)
    P4 = jnp.concatenate([Pa.reshape(2, NPAD, 16), Pb.reshape(2, NPAD, 16)], 0)

    h2, Sb = _mid(U4, P4, W_out, ab)

    ep2 = _make_edge_pass(kidx=None, epc=E // (2 * NS), tabn=NPAD)
    U2f, P2f = ep2(src, tgt, Gflat, Sb.reshape(-1), h2)
    U2 = U2f.reshape(2, NPAD, 64)
    P2 = P2f.reshape(2, NPAD, 16)

    return _final(U2, P2)[:N]
